# baseline (speedup 1.0000x reference)
import functools

import jax
import jax.numpy as jnp
from jax import lax
from jax.experimental import pallas as pl
from jax.experimental.pallas import tpu as pltpu

_FREQ_MASK_COUNT = 2
_FREQ_MASK_WIDTH = 8
_TIME_MASK_COUNT = 2
_TIME_MASK_WIDTH = 50
_TIME_MASK_RATIO = 0.1

_B, _T, _F = 128, 2000, 40
_ROW = _T * _F
_CB = 2
_NCH = _B // _CB
_NBUF = 8


def _mask_params(B, T, F):
    key = jax.random.key(42)
    kf_w, kf_s, kt_w, kt_s = jax.random.split(key, 4)
    max_time_mask = min(_TIME_MASK_WIDTH, int(T * _TIME_MASK_RATIO))

    f_width = jax.random.randint(kf_w, (B, _FREQ_MASK_COUNT), 0, _FREQ_MASK_WIDTH + 1)
    uf = jax.random.uniform(kf_s, (B, _FREQ_MASK_COUNT))
    f_hi = jnp.maximum(0, F - f_width - 1) + 1
    f_start = jnp.floor(uf * f_hi).astype(jnp.int32)

    t_width = jax.random.randint(kt_w, (B, _TIME_MASK_COUNT), 0, max(max_time_mask, 0) + 1)
    ut = jax.random.uniform(kt_s, (B, _TIME_MASK_COUNT))
    t_hi = jnp.maximum(0, T - t_width - 1) + 1
    t_start = jnp.floor(ut * t_hi).astype(jnp.int32)

    f_width = f_width.astype(jnp.int32)
    t_width = t_width.astype(jnp.int32)
    cols = [
        f_start[:, 0], f_start[:, 0] + f_width[:, 0],
        f_start[:, 1], f_start[:, 1] + f_width[:, 1],
        t_start[:, 0] * F, (t_start[:, 0] + t_width[:, 0]) * F,
        t_start[:, 1] * F, (t_start[:, 1] + t_width[:, 1]) * F,
    ]
    return jnp.stack(cols, axis=1)


def _tc_body(pb_hbm, x_hbm, o_hbm, ibuf, obuf, pbv, psem, isem, osem):
    pltpu.make_async_copy(pb_hbm, pbv, psem).start()

    def in_copy(i, slot):
        return pltpu.make_async_copy(
            x_hbm.at[pl.ds(i * _CB, _CB)], ibuf.at[slot], isem.at[slot])

    def out_copy(i, slot):
        return pltpu.make_async_copy(
            obuf.at[slot], o_hbm.at[pl.ds(i * _CB, _CB)], osem.at[slot])

    for k in range(_NBUF):
        in_copy(k, k).start()
    pltpu.make_async_copy(pb_hbm, pbv, psem).wait()

    def step(i, carry):
        slot = lax.rem(i, _NBUF)
        in_copy(i, slot).wait()

        @pl.when(i >= _NBUF)
        def _():
            out_copy(i - _NBUF, slot).wait()

        pb = pbv[pl.ds(i * _CB, _CB), :]
        x = ibuf[slot]
        li = lax.broadcasted_iota(jnp.int32, (_CB, _ROW), 1)
        t = (li.astype(jnp.float32) * (1.0 / _F)).astype(jnp.int32)
        f = li - t * _F

        def hit(v, lo, hi):
            return (v >= pb[:, lo:lo + 1]) & (v < pb[:, hi:hi + 1])

        masked = (hit(f, 0, 1) | hit(f, 2, 3)) | (hit(li, 4, 5) | hit(li, 6, 7))
        obuf[slot] = jnp.where(masked, 0.0, x)

        out_copy(i, slot).start()

        @pl.when(i + _NBUF < _NCH)
        def _():
            in_copy(i + _NBUF, slot).start()

        return carry

    lax.fori_loop(0, _NCH, step, 0)

    def drain(i, carry):
        j = _NCH - _NBUF + i
        out_copy(j, lax.rem(j, _NBUF)).wait()
        return carry

    lax.fori_loop(0, _NBUF, drain, 0)


@jax.jit
def _tc_apply(x2, params):
    return pl.pallas_call(
        _tc_body,
        in_specs=[
            pl.BlockSpec(memory_space=pl.ANY),
            pl.BlockSpec(memory_space=pl.ANY),
        ],
        out_specs=pl.BlockSpec(memory_space=pl.ANY),
        out_shape=jax.ShapeDtypeStruct((_B, _ROW), jnp.float32),
        scratch_shapes=[
            pltpu.VMEM((_NBUF, _CB, _ROW), jnp.float32),
            pltpu.VMEM((_NBUF, _CB, _ROW), jnp.float32),
            pltpu.VMEM((_B, 8), jnp.int32),
            pltpu.SemaphoreType.DMA,
            pltpu.SemaphoreType.DMA((_NBUF,)),
            pltpu.SemaphoreType.DMA((_NBUF,)),
        ],
    )(params, x2)


def kernel(x):
    B, T, F = x.shape
    params = _mask_params(B, T, F)
    out = _tc_apply(x.reshape(_B, _ROW), params)
    return out.reshape(B, T, F)


# speedup vs baseline: 2.2256x; 2.2256x over previous
import functools

import jax
import jax.numpy as jnp
from jax import lax
from jax.experimental import pallas as pl
from jax.experimental.pallas import tpu as pltpu

_FREQ_MASK_COUNT = 2
_FREQ_MASK_WIDTH = 8
_TIME_MASK_COUNT = 2
_TIME_MASK_WIDTH = 50
_TIME_MASK_RATIO = 0.1

_B, _T, _F = 128, 2000, 40
_ROW = _T * _F
_GB = 16
_G = _B // _GB


def _mask_params(B, T, F):
    key = jax.random.key(42)
    kf_w, kf_s, kt_w, kt_s = jax.random.split(key, 4)
    max_time_mask = min(_TIME_MASK_WIDTH, int(T * _TIME_MASK_RATIO))

    f_width = jax.random.randint(kf_w, (B, _FREQ_MASK_COUNT), 0, _FREQ_MASK_WIDTH + 1)
    uf = jax.random.uniform(kf_s, (B, _FREQ_MASK_COUNT))
    f_hi = jnp.maximum(0, F - f_width - 1) + 1
    f_start = jnp.floor(uf * f_hi).astype(jnp.int32)

    t_width = jax.random.randint(kt_w, (B, _TIME_MASK_COUNT), 0, max(max_time_mask, 0) + 1)
    ut = jax.random.uniform(kt_s, (B, _TIME_MASK_COUNT))
    t_hi = jnp.maximum(0, T - t_width - 1) + 1
    t_start = jnp.floor(ut * t_hi).astype(jnp.int32)

    f_width = f_width.astype(jnp.int32)
    t_width = t_width.astype(jnp.int32)
    cols = [
        f_start[:, 0], f_start[:, 0] + f_width[:, 0],
        f_start[:, 1], f_start[:, 1] + f_width[:, 1],
        t_start[:, 0] * F, (t_start[:, 0] + t_width[:, 0]) * F,
        t_start[:, 1] * F, (t_start[:, 1] + t_width[:, 1]) * F,
    ]
    return jnp.stack(cols, axis=1)


def _tc_body(pb_ref, x_ref, o_ref):
    pb = pb_ref[...]
    x = x_ref[...]

    li = lax.broadcasted_iota(jnp.int32, (_GB, _ROW), 1)
    t = (li.astype(jnp.float32) * (1.0 / _F)).astype(jnp.int32)
    f = li - t * _F

    def hit(v, lo, hi):
        return (v >= pb[:, lo:lo + 1]) & (v < pb[:, hi:hi + 1])

    masked = (hit(f, 0, 1) | hit(f, 2, 3)) | (hit(li, 4, 5) | hit(li, 6, 7))
    o_ref[...] = jnp.where(masked, 0.0, x)


@jax.jit
def _tc_apply(x2, params):
    return pl.pallas_call(
        _tc_body,
        grid=(_G,),
        in_specs=[
            pl.BlockSpec((_GB, 8), lambda i: (i, 0)),
            pl.BlockSpec((_GB, _ROW), lambda i: (i, 0)),
        ],
        out_specs=pl.BlockSpec((_GB, _ROW), lambda i: (i, 0)),
        out_shape=jax.ShapeDtypeStruct((_B, _ROW), jnp.float32),
    )(params, x2)


def kernel(x):
    B, T, F = x.shape
    params = _mask_params(B, T, F)
    out = _tc_apply(x.reshape(_B, _ROW), params)
    return out.reshape(B, T, F)


# speedup vs baseline: 2.2424x; 1.0075x over previous
import functools

import jax
import jax.numpy as jnp
from jax import lax
from jax.experimental import pallas as pl
from jax.experimental.pallas import tpu as pltpu

_FREQ_MASK_COUNT = 2
_FREQ_MASK_WIDTH = 8
_TIME_MASK_COUNT = 2
_TIME_MASK_WIDTH = 50
_TIME_MASK_RATIO = 0.1

_B, _T, _F = 128, 2000, 40
_ROW = _T * _F
_GB = 32
_G = _B // _GB


def _mask_params(B, T, F):
    key = jax.random.key(42)
    kf_w, kf_s, kt_w, kt_s = jax.random.split(key, 4)
    max_time_mask = min(_TIME_MASK_WIDTH, int(T * _TIME_MASK_RATIO))

    f_width = jax.random.randint(kf_w, (B, _FREQ_MASK_COUNT), 0, _FREQ_MASK_WIDTH + 1)
    uf = jax.random.uniform(kf_s, (B, _FREQ_MASK_COUNT))
    f_hi = jnp.maximum(0, F - f_width - 1) + 1
    f_start = jnp.floor(uf * f_hi).astype(jnp.int32)

    t_width = jax.random.randint(kt_w, (B, _TIME_MASK_COUNT), 0, max(max_time_mask, 0) + 1)
    ut = jax.random.uniform(kt_s, (B, _TIME_MASK_COUNT))
    t_hi = jnp.maximum(0, T - t_width - 1) + 1
    t_start = jnp.floor(ut * t_hi).astype(jnp.int32)

    f_width = f_width.astype(jnp.int32)
    t_width = t_width.astype(jnp.int32)
    cols = [
        f_start[:, 0], f_start[:, 0] + f_width[:, 0],
        f_start[:, 1], f_start[:, 1] + f_width[:, 1],
        t_start[:, 0] * F, (t_start[:, 0] + t_width[:, 0]) * F,
        t_start[:, 1] * F, (t_start[:, 1] + t_width[:, 1]) * F,
    ]
    return jnp.stack(cols, axis=1)


def _tc_body(pb_ref, x_ref, o_ref):
    pb = pb_ref[...]
    x = x_ref[...]

    li = lax.broadcasted_iota(jnp.int32, (_GB, _ROW), 1)
    t = (li.astype(jnp.float32) * (1.0 / _F)).astype(jnp.int32)
    f = li - t * _F

    def hit(v, lo, hi):
        return (v >= pb[:, lo:lo + 1]) & (v < pb[:, hi:hi + 1])

    masked = (hit(f, 0, 1) | hit(f, 2, 3)) | (hit(li, 4, 5) | hit(li, 6, 7))
    o_ref[...] = jnp.where(masked, 0.0, x)


@jax.jit
def _tc_apply(x2, params):
    return pl.pallas_call(
        _tc_body,
        grid=(_G,),
        in_specs=[
            pl.BlockSpec((_GB, 8), lambda i: (i, 0)),
            pl.BlockSpec((_GB, _ROW), lambda i: (i, 0)),
        ],
        out_specs=pl.BlockSpec((_GB, _ROW), lambda i: (i, 0)),
        out_shape=jax.ShapeDtypeStruct((_B, _ROW), jnp.float32),
    )(params, x2)


def kernel(x):
    B, T, F = x.shape
    params = _mask_params(B, T, F)
    out = _tc_apply(x.reshape(_B, _ROW), params)
    return out.reshape(B, T, F)
